# Initial kernel scaffold; baseline (speedup 1.0000x reference)
import jax
import jax.numpy as jnp
from jax.experimental import pallas as pl
from jax.experimental.pallas import tpu as pltpu

_T = 32


def _h_body(x_ref, o_ref):
    d = pl.program_id(0)
    a = x_ref[0]
    b = jnp.where(d == 1, jnp.flip(a, axis=0), a)
    rows = jax.lax.broadcasted_iota(jnp.int32, (a.shape[0], 1, 1), 0)
    o_ref[0] = jnp.where(rows % 2 == 1, jnp.flip(b, axis=1), b)


def _v_body(x_ref, _prev_ref, o_ref):
    d = pl.program_id(0)
    a = x_ref[0]
    t = jnp.swapaxes(a, 0, 1)
    b = jnp.where(d == 1, jnp.flip(t, axis=0), t)
    cols = jax.lax.broadcasted_iota(jnp.int32, (t.shape[0], 1, 1), 0)
    o_ref[0] = jnp.where(cols % 2 == 1, jnp.flip(b, axis=1), b)


def kernel(x_2d):
    N, H, W, C = x_2d.shape
    L = H * W
    T = _T
    HT = H // T
    WT = W // T

    out_shape = jax.ShapeDtypeStruct((4 * N, H, W, C), x_2d.dtype)

    h_out = pl.pallas_call(
        _h_body,
        grid=(2, N, HT),
        in_specs=[
            pl.BlockSpec(
                (1, T, W, C),
                lambda d, n, rt: (n, jnp.where(d == 1, HT - 1 - rt, rt), 0, 0),
            )
        ],
        out_specs=pl.BlockSpec(
            (1, T, W, C),
            lambda d, n, rt: (d * N + n, rt, 0, 0),
        ),
        out_shape=out_shape,
    )(x_2d)

    full = pl.pallas_call(
        _v_body,
        grid=(2, N, WT),
        in_specs=[
            pl.BlockSpec(
                (1, H, T, C),
                lambda d, n, ct: (n, 0, jnp.where(d == 1, WT - 1 - ct, ct), 0),
            ),
            pl.BlockSpec(memory_space=pltpu.ANY),
        ],
        out_specs=pl.BlockSpec(
            (1, T, H, C),
            lambda d, n, ct: (2 * N + d * N + n, ct, 0, 0),
        ),
        out_shape=out_shape,
        input_output_aliases={1: 0},
    )(x_2d, h_out)

    return full.reshape(4 * N, L, C)


# speedup vs baseline: 1.5154x; 1.5154x over previous
import jax
import jax.numpy as jnp
from jax.experimental import pallas as pl

_T = 32


def _rev_sublanes(a):
    T, S, C = a.shape
    g = S // 8
    a4 = a.reshape(T, g, 8, C)
    sidx = jax.lax.broadcasted_iota(jnp.int32, (T, g, 8, C), 2)
    within = jnp.take_along_axis(a4, 7 - sidx, axis=2)
    out = jnp.concatenate(
        [within[:, g - 1 - i:g - i] for i in range(g)], axis=1)
    return out.reshape(T, S, C)


def _rev_major(a):
    T = a.shape[0]
    return jnp.concatenate([a[T - 1 - i:T - i] for i in range(T)], axis=0)


def _h_body(x_ref, o_ref):
    d = pl.program_id(0)
    a = x_ref[0]
    rev = _rev_sublanes(a)
    odd = jax.lax.broadcasted_iota(jnp.int32, (a.shape[0], 1, 1), 0) % 2 == 1

    @pl.when(d == 0)
    def _():
        o_ref[0] = jnp.where(odd, rev, a)

    @pl.when(d == 1)
    def _():
        o_ref[0] = _rev_major(jnp.where(odd, a, rev))


def _v_body(x_ref, _prev_ref, o_ref):
    d = pl.program_id(0)
    a = x_ref[0]
    t = jnp.swapaxes(a, 0, 1)
    rev = _rev_sublanes(t)
    odd = jax.lax.broadcasted_iota(jnp.int32, (t.shape[0], 1, 1), 0) % 2 == 1

    @pl.when(d == 0)
    def _():
        o_ref[0] = jnp.where(odd, rev, t)

    @pl.when(d == 1)
    def _():
        o_ref[0] = _rev_major(jnp.where(odd, t, rev))


def kernel(x_2d):
    N, H, W, C = x_2d.shape
    L = H * W
    T = _T
    HT = H // T
    WT = W // T

    out_shape = jax.ShapeDtypeStruct((4 * N, H, W, C), x_2d.dtype)

    h_out = pl.pallas_call(
        _h_body,
        grid=(2, N, HT),
        in_specs=[
            pl.BlockSpec(
                (1, T, W, C),
                lambda d, n, rt: (n, jnp.where(d == 1, HT - 1 - rt, rt), 0, 0),
            )
        ],
        out_specs=pl.BlockSpec(
            (1, T, W, C),
            lambda d, n, rt: (d * N + n, rt, 0, 0),
        ),
        out_shape=out_shape,
    )(x_2d)

    full = pl.pallas_call(
        _v_body,
        grid=(2, N, WT),
        in_specs=[
            pl.BlockSpec(
                (1, H, T, C),
                lambda d, n, ct: (n, 0, jnp.where(d == 1, WT - 1 - ct, ct), 0),
            ),
            pl.BlockSpec(memory_space=pl.ANY),
        ],
        out_specs=pl.BlockSpec(
            (1, T, H, C),
            lambda d, n, ct: (2 * N + d * N + n, ct, 0, 0),
        ),
        out_shape=out_shape,
        input_output_aliases={1: 0},
    )(x_2d, h_out)

    return full.reshape(4 * N, L, C)


# speedup vs baseline: 1.8072x; 1.1925x over previous
import jax
import jax.numpy as jnp
from jax.experimental import pallas as pl
from jax.experimental.pallas import tpu as pltpu

_T = 8


def _rev_sublanes_wide(a):
    T, S, C = a.shape
    g = S // 8
    a4 = a.reshape(T, g, 8, C)
    sidx = jax.lax.broadcasted_iota(jnp.int32, (T, g, 8, C), 2)
    within = jnp.take_along_axis(a4, 7 - sidx, axis=2)
    out = jnp.concatenate(
        [within[:, g - 1 - i:g - i] for i in range(g)], axis=1)
    return out.reshape(T, S, C)


def _rev_sublanes8(a):
    sidx = jax.lax.broadcasted_iota(jnp.int32, a.shape, 1)
    return jnp.take_along_axis(a, 7 - sidx, axis=1)


def _rev_major(a):
    T = a.shape[0]
    return jnp.concatenate([a[T - 1 - i:T - i] for i in range(T)], axis=0)


def _body(xa_ref, xb_ref, o_ref, hstage, vstage, sems):
    N, H, W, C = o_ref.shape[0] // 4, o_ref.shape[1], o_ref.shape[2], o_ref.shape[3]
    T = _T
    n = pl.program_id(0)
    i = pl.program_id(1)
    npairs = pl.num_programs(1)
    k = n * npairs + i
    last = (pl.num_programs(0) * npairs) - 1
    sl = jax.lax.rem(k, 2)

    A = xa_ref[0]
    B = xb_ref[0]
    Ar = _rev_sublanes_wide(A)
    Br = _rev_sublanes_wide(B)
    odd_t = jax.lax.broadcasted_iota(jnp.int32, (T, 1, 1), 0) % 2 == 1
    odd_c = jax.lax.broadcasted_iota(jnp.int32, (W, 1, 1), 0) % 2 == 1

    At = jnp.swapaxes(A, 0, 1)
    Bt = jnp.swapaxes(B, 0, 1)
    Art = jnp.swapaxes(Ar, 0, 1)
    Brt = jnp.swapaxes(Br, 0, 1)

    def _mk(stage, si, sj, dst, slot=None):
        s = sl if slot is None else slot
        return pltpu.make_async_copy(stage.at[s, si], dst, sems.at[s, sj])

    row_lo = pl.ds(i * T, T)
    row_hi = pl.ds(H - (i + 1) * T, T)
    dsts = [
        o_ref.at[n, row_lo],
        o_ref.at[n, row_hi],
        o_ref.at[N + n, row_lo],
        o_ref.at[N + n, row_hi],
    ]
    vdsts = [
        o_ref.at[2 * N + n, :, row_lo],
        o_ref.at[2 * N + n, :, row_hi],
        o_ref.at[3 * N + n, :, row_lo],
        o_ref.at[3 * N + n, :, row_hi],
    ]

    @pl.when(k >= 2)
    def _():
        for j in range(4):
            _mk(hstage, j, j, dsts[j]).wait()
            _mk(vstage, j, j + 4, vdsts[j]).wait()

    hstage[sl, 0] = jnp.where(odd_t, Ar, A)
    hstage[sl, 1] = jnp.where(odd_t, Br, B)
    hstage[sl, 2] = _rev_major(jnp.where(odd_t, B, Br))
    hstage[sl, 3] = _rev_major(jnp.where(odd_t, A, Ar))
    vstage[sl, 0] = jnp.where(odd_c, _rev_sublanes8(Bt), At)
    vstage[sl, 1] = jnp.where(odd_c, _rev_sublanes8(At), Bt)
    vstage[sl, 2] = jnp.where(odd_c, _rev_sublanes8(Brt), Art)
    vstage[sl, 3] = jnp.where(odd_c, _rev_sublanes8(Art), Brt)

    for j in range(4):
        _mk(hstage, j, j, dsts[j]).start()
        _mk(vstage, j, j + 4, vdsts[j]).start()

    @pl.when(k == last)
    def _():
        for j in range(4):
            _mk(hstage, j, j, dsts[j]).wait()
            _mk(vstage, j, j + 4, vdsts[j]).wait()
            _mk(hstage, j, j, dsts[j], slot=1 - sl).wait()
            _mk(vstage, j, j + 4, vdsts[j], slot=1 - sl).wait()


def kernel(x_2d):
    N, H, W, C = x_2d.shape
    L = H * W
    T = _T
    npairs = H // (2 * T)

    out = pl.pallas_call(
        _body,
        grid=(N, npairs),
        in_specs=[
            pl.BlockSpec((1, T, W, C), lambda n, i: (n, i, 0, 0)),
            pl.BlockSpec(
                (1, T, W, C),
                lambda n, i: (n, (H // _T) - 1 - i, 0, 0)),
        ],
        out_specs=pl.BlockSpec(memory_space=pl.ANY),
        out_shape=jax.ShapeDtypeStruct((4 * N, H, W, C), x_2d.dtype),
        scratch_shapes=[
            pltpu.VMEM((2, 4, T, W, C), x_2d.dtype),
            pltpu.VMEM((2, 4, W, T, C), x_2d.dtype),
            pltpu.SemaphoreType.DMA((2, 8)),
        ],
    )(x_2d, x_2d)

    return out.reshape(4 * N, L, C)
